# baseline (speedup 1.0000x reference)
import functools

import jax
import jax.numpy as jnp
from jax import lax
from jax.experimental import pallas as pl
from jax.experimental.pallas import tpu as pltpu
from jax.experimental.pallas import tpu_sc as plsc

NUM_EMB = 1_000_000
DIM = 64
BATCH = 4096
SEQ = 200
NTOK = BATCH * SEQ

NC = 2
NS = 16
NW = NC * NS

ABLK = 128
NBLK_A = 7813
TBL_PAD = NBLK_A * ABLK

BT = BATCH // 128


def _transpose_body(embt_hbm, tlin_hbm, in_v, out_v, isem, osem):
    wid = lax.axis_index("s") * NC + lax.axis_index("c")
    k16 = lax.iota(jnp.int32, 16)
    dt_idx = [(b * 16 + k16) // 8 for b in range(4)]
    di_idx = [(b * 16 + k16) % 8 for b in range(4)]
    nj = (NBLK_A - wid + NW - 1) // NW

    def fire_in(i, slot):
        j = pl.multiple_of((wid + i * NW) * ABLK, ABLK)
        for dt in range(8):
            pltpu.async_copy(
                embt_hbm.at[pl.ds(dt * 8, 8), pl.ds(j, ABLK)],
                in_v.at[slot, dt], isem.at[slot])

    def wait_in(i, slot):
        j = pl.multiple_of((wid + i * NW) * ABLK, ABLK)
        for dt in range(8):
            pltpu.make_async_copy(
                embt_hbm.at[pl.ds(dt * 8, 8), pl.ds(j, ABLK)],
                in_v.at[slot, dt], isem.at[slot]).wait()

    def fire_out(i, slot):
        j = wid + i * NW
        pltpu.async_copy(
            out_v.at[slot], tlin_hbm.at[pl.ds(j * ABLK * DIM, ABLK * DIM)],
            osem.at[slot])

    def wait_out(i, slot):
        j = wid + i * NW
        pltpu.make_async_copy(
            out_v.at[slot], tlin_hbm.at[pl.ds(j * ABLK * DIM, ABLK * DIM)],
            osem.at[slot]).wait()

    @pl.when(nj > 0)
    def _():
        fire_in(0, 0)

        def step(i, carry):
            slot = lax.rem(i, 2)

            @pl.when(i + 1 < nj)
            def _():
                fire_in(i + 1, 1 - slot)

            wait_in(i, slot)

            @pl.when(i >= 2)
            def _():
                wait_out(i - 2, slot)

            sslot = jnp.full((16,), slot, jnp.int32)

            def shuf(r, c):
                rv = jnp.full((16,), r, jnp.int32)
                for b in range(4):
                    v = plsc.load_gather(in_v, [sslot, dt_idx[b], di_idx[b], rv])
                    out_v[slot, pl.ds(r * DIM + b * 16, 16)] = v
                return c

            lax.fori_loop(0, ABLK, shuf, 0)
            fire_out(i, slot)
            return carry

        lax.fori_loop(0, nj, step, 0)

        @pl.when(nj >= 2)
        def _():
            wait_out(nj - 2, lax.rem(nj - 2, 2))

        wait_out(nj - 1, lax.rem(nj - 1, 2))


def _gather_body(ids_hbm, table_hbm, out_hbm, idx_v, g_v, t_v, xsem, gsem, ssem):
    wid = lax.axis_index("s") * NC + lax.axis_index("c")
    k16 = lax.iota(jnp.int32, 16)
    bi_idx = [(b * 16 + k16) for b in range(8)]

    pltpu.async_copy(
        ids_hbm.at[pl.ds(0, SEQ), pl.ds(wid * 128, 128)], idx_v, xsem)
    pltpu.make_async_copy(
        ids_hbm.at[pl.ds(0, SEQ), pl.ds(wid * 128, 128)], idx_v, xsem).wait()

    def fire_g(s, slot):
        pltpu.async_copy(table_hbm.at[idx_v.at[s]], g_v.at[slot], gsem.at[slot])

    def wait_g(s, slot):
        pltpu.make_async_copy(
            table_hbm.at[idx_v.at[s]], g_v.at[slot], gsem.at[slot]).wait()

    def fire_s(s, slot):
        for dt in range(8):
            pltpu.async_copy(
                t_v.at[slot, pl.ds(dt * 8, 8)], out_hbm.at[s, dt, wid],
                ssem.at[slot])

    def wait_s(s, slot):
        for dt in range(8):
            pltpu.make_async_copy(
                t_v.at[slot, pl.ds(dt * 8, 8)], out_hbm.at[s, dt, wid],
                ssem.at[slot]).wait()

    fire_g(0, 0)

    def step(s, carry):
        slot = lax.rem(s, 2)

        @pl.when(s + 1 < SEQ)
        def _():
            fire_g(s + 1, 1 - slot)

        wait_g(s, slot)

        @pl.when(s >= 2)
        def _():
            wait_s(s - 2, slot)

        sslot = jnp.full((16,), slot, jnp.int32)

        def shuf(d, c):
            dv = jnp.full((16,), d, jnp.int32)
            for b in range(8):
                v = plsc.load_gather(g_v, [sslot, bi_idx[b], dv])
                t_v[slot, d, pl.ds(b * 16, 16)] = v
            return c

        lax.fori_loop(0, DIM, shuf, 0)
        fire_s(s, slot)
        return carry

    lax.fori_loop(0, SEQ, step, 0)
    wait_s(SEQ - 2, 0)
    wait_s(SEQ - 1, 1)


def _make_transpose_kernel():
    mesh = plsc.VectorSubcoreMesh(core_axis_name="c", subcore_axis_name="s")
    return pl.kernel(
        _transpose_body,
        out_type=jax.ShapeDtypeStruct((TBL_PAD * DIM,), jnp.float32),
        mesh=mesh,
        scratch_types=[
            pltpu.VMEM((2, 8, 8, ABLK), jnp.float32),
            pltpu.VMEM((2, ABLK * DIM), jnp.float32),
            pltpu.SemaphoreType.DMA((2,)),
            pltpu.SemaphoreType.DMA((2,)),
        ],
        compiler_params=pltpu.CompilerParams(
            use_tc_tiling_on_sc=True, needs_layout_passes=False,
            disable_bounds_checks=True),
    )


def _make_gather_kernel():
    mesh = plsc.VectorSubcoreMesh(core_axis_name="c", subcore_axis_name="s")
    return pl.kernel(
        _gather_body,
        out_type=jax.ShapeDtypeStruct((SEQ, 8, BT, 8, 128), jnp.float32),
        mesh=mesh,
        scratch_types=[
            pltpu.VMEM((SEQ, 128), jnp.int32),
            pltpu.VMEM((2, 128, DIM), jnp.float32),
            pltpu.VMEM((2, DIM, 128), jnp.float32),
            pltpu.SemaphoreType.DMA,
            pltpu.SemaphoreType.DMA((2,)),
            pltpu.SemaphoreType.DMA((2,)),
        ],
        compiler_params=pltpu.CompilerParams(
            use_tc_tiling_on_sc=False, needs_layout_passes=False),
    )


@jax.jit
def _emb_call(ids_t, embt):
    tlin = _make_transpose_kernel()(embt)
    table = tlin.reshape(TBL_PAD, DIM)
    return _make_gather_kernel()(ids_t, table)


def kernel(token_ids, embedding):
    embt = embedding.T
    ids_t = token_ids.T.astype(jnp.int32)
    out5d = _emb_call(ids_t, embt)
    return out5d.transpose(2, 4, 0, 1, 3).reshape(BATCH, SEQ, DIM)


# speedup vs baseline: 1.0079x; 1.0079x over previous
import jax
import jax.numpy as jnp
from jax import lax
from jax.experimental import pallas as pl
from jax.experimental.pallas import tpu as pltpu
from jax.experimental.pallas import tpu_sc as plsc

NUM_EMB = 1_000_000
DIM = 64
BATCH = 4096
SEQ = 200

NC = 2
NS = 16
NW = NC * NS

NBLK_A = 7813
TBL_PAD = NBLK_A * 128
BT = BATCH // 128


def _transpose_body(embt_hbm, tlin_hbm, in_v, out_v, isem, osem):
    wid = lax.axis_index("s") * NC + lax.axis_index("c")
    k16 = lax.iota(jnp.int32, 16)
    nj = (NBLK_A - wid + NW - 1) // NW

    def fire_in(i, slot):
        j = pl.multiple_of((wid + i * NW) * 128, 128)
        pltpu.async_copy(embt_hbm.at[pl.ds(0, DIM), pl.ds(j, 128)],
                         in_v.at[slot], isem.at[slot])

    def wait_in(i, slot):
        j = pl.multiple_of((wid + i * NW) * 128, 128)
        pltpu.make_async_copy(embt_hbm.at[pl.ds(0, DIM), pl.ds(j, 128)],
                              in_v.at[slot], isem.at[slot]).wait()

    def fire_out(i, slot):
        j = wid + i * NW
        pltpu.async_copy(out_v.at[slot],
                         tlin_hbm.at[pl.ds(j * 128 * DIM, 128 * DIM)],
                         osem.at[slot])

    def wait_out(i, slot):
        j = wid + i * NW
        pltpu.make_async_copy(out_v.at[slot],
                              tlin_hbm.at[pl.ds(j * 128 * DIM, 128 * DIM)],
                              osem.at[slot]).wait()

    sconst = [jnp.full((16,), b, jnp.int32) for b in range(2)]
    dconst = [db * 16 + k16 for db in range(4)]

    def shuffle(slot):
        def shuf(r, c):
            rv = jnp.full((16,), r, jnp.int32)
            for db in range(4):
                v = plsc.load_gather(in_v, [sconst[slot], dconst[db], rv])
                out_v[slot, pl.ds(r * DIM + db * 16, 16)] = v
            return c
        lax.fori_loop(0, 128, shuf, 0, unroll=2)

    fire_in(0, 0)
    ni2 = (nj + 1) // 2

    def step(i2, carry):
        for b in range(2):
            i = 2 * i2 + b

            @pl.when(i < nj)
            def _():
                @pl.when(i + 1 < nj)
                def _():
                    fire_in(i + 1, 1 - b)

                wait_in(i, b)

                @pl.when(i >= 2)
                def _():
                    wait_out(i - 2, b)

                shuffle(b)
                fire_out(i, b)

        return carry

    lax.fori_loop(0, ni2, step, 0)

    @pl.when(nj >= 2)
    def _():
        wait_out(nj - 2, lax.rem(nj - 2, 2))

    wait_out(nj - 1, lax.rem(nj - 1, 2))


def _gather_body(ids_hbm, table_hbm, out_hbm, idx_v, g_v, t_v, xsem, gsem, ssem):
    wid = lax.axis_index("s") * NC + lax.axis_index("c")
    k16 = lax.iota(jnp.int32, 16)

    pltpu.async_copy(ids_hbm.at[pl.ds(0, SEQ), pl.ds(wid * 128, 128)],
                     idx_v, xsem)
    pltpu.make_async_copy(ids_hbm.at[pl.ds(0, SEQ), pl.ds(wid * 128, 128)],
                          idx_v, xsem).wait()

    def fire_g(s, slot):
        pltpu.async_copy(table_hbm.at[idx_v.at[s]], g_v.at[slot], gsem.at[slot])

    def wait_g(s, slot):
        pltpu.make_async_copy(table_hbm.at[idx_v.at[s]], g_v.at[slot],
                              gsem.at[slot]).wait()

    def fire_s(s, slot):
        pltpu.async_copy(t_v.at[slot], out_hbm.at[s, pl.ds(0, 8), pl.ds(wid, 1)],
                         ssem.at[slot])

    def wait_s(s, slot):
        pltpu.make_async_copy(t_v.at[slot],
                              out_hbm.at[s, pl.ds(0, 8), pl.ds(wid, 1)],
                              ssem.at[slot]).wait()

    sconst = [jnp.full((16,), b, jnp.int32) for b in range(2)]
    bconst = [bb * 16 + k16 for bb in range(8)]

    def shuffle(slot):
        def shuf(d, c):
            dv = jnp.full((16,), d, jnp.int32)
            dt = d // 8
            di = lax.rem(d, 8)
            for bb in range(8):
                v = plsc.load_gather(g_v, [sconst[slot], bconst[bb], dv])
                t_v[slot, dt, 0, di, pl.ds(bb * 16, 16)] = v
            return c
        lax.fori_loop(0, DIM, shuf, 0, unroll=2)

    fire_g(0, 0)
    ns2 = SEQ // 2

    def step(s2, carry):
        for b in range(2):
            s = 2 * s2 + b

            @pl.when(s + 1 < SEQ)
            def _():
                fire_g(s + 1, 1 - b)

            wait_g(s, b)

            @pl.when(s >= 2)
            def _():
                wait_s(s - 2, b)

            shuffle(b)
            fire_s(s, b)
        return carry

    lax.fori_loop(0, ns2, step, 0)
    wait_s(SEQ - 2, 0)
    wait_s(SEQ - 1, 1)


def _make_transpose_kernel():
    mesh = plsc.VectorSubcoreMesh(core_axis_name="c", subcore_axis_name="s")
    return pl.kernel(
        _transpose_body,
        out_type=jax.ShapeDtypeStruct((TBL_PAD * DIM,), jnp.float32),
        mesh=mesh,
        scratch_types=[
            pltpu.VMEM((2, DIM, 128), jnp.float32),
            pltpu.VMEM((2, 128 * DIM), jnp.float32),
            pltpu.SemaphoreType.DMA((2,)),
            pltpu.SemaphoreType.DMA((2,)),
        ],
        compiler_params=pltpu.CompilerParams(
            use_tc_tiling_on_sc=True, needs_layout_passes=False,
            disable_bounds_checks=True),
    )


def _make_gather_kernel():
    mesh = plsc.VectorSubcoreMesh(core_axis_name="c", subcore_axis_name="s")
    return pl.kernel(
        _gather_body,
        out_type=jax.ShapeDtypeStruct((SEQ, 8, BT, 8, 128), jnp.float32),
        mesh=mesh,
        scratch_types=[
            pltpu.VMEM((SEQ, 128), jnp.int32),
            pltpu.VMEM((2, 128, DIM), jnp.float32),
            pltpu.VMEM((2, 8, 1, 8, 128), jnp.float32),
            pltpu.SemaphoreType.DMA,
            pltpu.SemaphoreType.DMA((2,)),
            pltpu.SemaphoreType.DMA((2,)),
        ],
        compiler_params=pltpu.CompilerParams(
            use_tc_tiling_on_sc=False, needs_layout_passes=False),
    )


@jax.jit
def _emb_call(ids_t, embt):
    tlin = _make_transpose_kernel()(embt)
    table = tlin.reshape(TBL_PAD, DIM)
    return _make_gather_kernel()(ids_t, table)


def kernel(token_ids, embedding):
    embt = embedding.T
    ids_t = token_ids.T.astype(jnp.int32)
    out5d = _emb_call(ids_t, embt)
    return out5d.transpose(2, 4, 0, 1, 3).reshape(BATCH, SEQ, DIM)


# speedup vs baseline: 1.6399x; 1.6271x over previous
import jax
import jax.numpy as jnp
from jax import lax
from jax.experimental import pallas as pl
from jax.experimental.pallas import tpu as pltpu
from jax.experimental.pallas import tpu_sc as plsc

NUM_EMB = 1_000_000
DIM = 64
BATCH = 4096
SEQ = 200

NC = 2
NS = 16
NW = NC * NS

NBLK_A = 7813
TBL_PAD = NBLK_A * 128
BT = BATCH // 128


def _transpose_body(embt_hbm, tlin_hbm, in_v, out_v, isem, osem):
    wid = lax.axis_index("s") * NC + lax.axis_index("c")
    k16 = lax.iota(jnp.int32, 16)
    nj = (NBLK_A - wid + NW - 1) // NW

    def fire_in(i, slot):
        j = pl.multiple_of((wid + i * NW) * 128, 128)
        pltpu.async_copy(embt_hbm.at[pl.ds(0, DIM), pl.ds(j, 128)],
                         in_v.at[slot], isem.at[slot])

    def wait_in(i, slot):
        j = pl.multiple_of((wid + i * NW) * 128, 128)
        pltpu.make_async_copy(embt_hbm.at[pl.ds(0, DIM), pl.ds(j, 128)],
                              in_v.at[slot], isem.at[slot]).wait()

    def fire_out(i, slot):
        j = wid + i * NW
        pltpu.async_copy(out_v.at[slot],
                         tlin_hbm.at[pl.ds(j * 128 * DIM, 128 * DIM)],
                         osem.at[slot])

    def wait_out(i, slot):
        j = wid + i * NW
        pltpu.make_async_copy(out_v.at[slot],
                              tlin_hbm.at[pl.ds(j * 128 * DIM, 128 * DIM)],
                              osem.at[slot]).wait()

    sconst = [jnp.full((16,), b, jnp.int32) for b in range(2)]
    din_c = [[d0 + (k16 + s) % 16 for s in range(16)] for d0 in (0, 16, 32, 48)]
    oix_c = [[k16 * DIM + d0 + (k16 + s) % 16 for s in range(16)]
             for d0 in (0, 16, 32, 48)]

    def shuffle(slot):
        def shuf(rb, c):
            rvec = rb * 16 + k16
            r064 = rb * (16 * DIM)
            for d0 in range(4):
                for s in range(16):
                    v = plsc.load_gather(in_v, [sconst[slot], din_c[d0][s], rvec])
                    plsc.store_scatter(out_v, [sconst[slot], oix_c[d0][s] + r064], v)
            return c
        lax.fori_loop(0, 8, shuf, 0)

    fire_in(0, 0)
    ni2 = (nj + 1) // 2

    def step(i2, carry):
        for b in range(2):
            i = 2 * i2 + b

            @pl.when(i < nj)
            def _():
                @pl.when(i + 1 < nj)
                def _():
                    fire_in(i + 1, 1 - b)

                wait_in(i, b)

                @pl.when(i >= 2)
                def _():
                    wait_out(i - 2, b)

                shuffle(b)
                fire_out(i, b)

        return carry

    lax.fori_loop(0, ni2, step, 0)

    @pl.when(nj >= 2)
    def _():
        wait_out(nj - 2, lax.rem(nj - 2, 2))

    wait_out(nj - 1, lax.rem(nj - 1, 2))


def _gather_body(ids_hbm, table_hbm, out_hbm, idx_v, g_v, t_v, xsem, gsem, ssem):
    wid = lax.axis_index("s") * NC + lax.axis_index("c")
    k16 = lax.iota(jnp.int32, 16)

    pltpu.async_copy(ids_hbm.at[pl.ds(0, SEQ), pl.ds(wid * 128, 128)],
                     idx_v, xsem)
    pltpu.make_async_copy(ids_hbm.at[pl.ds(0, SEQ), pl.ds(wid * 128, 128)],
                          idx_v, xsem).wait()

    def fire_g(s, slot):
        pltpu.async_copy(table_hbm.at[idx_v.at[s]], g_v.at[slot], gsem.at[slot])

    def wait_g(s, slot):
        pltpu.make_async_copy(table_hbm.at[idx_v.at[s]], g_v.at[slot],
                              gsem.at[slot]).wait()

    def fire_s(s, slot):
        pltpu.async_copy(t_v.at[slot], out_hbm.at[s, pl.ds(0, 8), pl.ds(wid, 1)],
                         ssem.at[slot])

    def wait_s(s, slot):
        pltpu.make_async_copy(t_v.at[slot],
                              out_hbm.at[s, pl.ds(0, 8), pl.ds(wid, 1)],
                              ssem.at[slot]).wait()

    sconst = [jnp.full((16,), b, jnp.int32) for b in range(2)]
    zconst = jnp.full((16,), 0, jnp.int32)
    dvec_c = [[d0 + (k16 + s) % 16 for s in range(16)] for d0 in (0, 16, 32, 48)]
    dt_c = [[v // 8 for v in row] for row in dvec_c]
    di_c = [[v % 8 for v in row] for row in dvec_c]

    def shuffle(slot):
        def shuf(bb, c):
            bivec = bb * 16 + k16
            for d0 in range(4):
                for s in range(16):
                    v = plsc.load_gather(g_v, [sconst[slot], bivec, dvec_c[d0][s]])
                    plsc.store_scatter(
                        t_v, [sconst[slot], dt_c[d0][s], zconst, di_c[d0][s], bivec], v)
            return c
        lax.fori_loop(0, 8, shuf, 0)

    fire_g(0, 0)
    ns2 = SEQ // 2

    def step(s2, carry):
        for b in range(2):
            s = 2 * s2 + b

            @pl.when(s + 1 < SEQ)
            def _():
                fire_g(s + 1, 1 - b)

            wait_g(s, b)

            @pl.when(s >= 2)
            def _():
                wait_s(s - 2, b)

            shuffle(b)
            fire_s(s, b)
        return carry

    lax.fori_loop(0, ns2, step, 0)
    wait_s(SEQ - 2, 0)
    wait_s(SEQ - 1, 1)


def _make_transpose_kernel():
    mesh = plsc.VectorSubcoreMesh(core_axis_name="c", subcore_axis_name="s")
    return pl.kernel(
        _transpose_body,
        out_type=jax.ShapeDtypeStruct((TBL_PAD * DIM,), jnp.float32),
        mesh=mesh,
        scratch_types=[
            pltpu.VMEM((2, DIM, 128), jnp.float32),
            pltpu.VMEM((2, 128 * DIM), jnp.float32),
            pltpu.SemaphoreType.DMA((2,)),
            pltpu.SemaphoreType.DMA((2,)),
        ],
        compiler_params=pltpu.CompilerParams(
            use_tc_tiling_on_sc=True, needs_layout_passes=False,
            disable_bounds_checks=True),
    )


def _make_gather_kernel():
    mesh = plsc.VectorSubcoreMesh(core_axis_name="c", subcore_axis_name="s")
    return pl.kernel(
        _gather_body,
        out_type=jax.ShapeDtypeStruct((SEQ, 8, BT, 8, 128), jnp.float32),
        mesh=mesh,
        scratch_types=[
            pltpu.VMEM((SEQ, 128), jnp.int32),
            pltpu.VMEM((2, 128, DIM), jnp.float32),
            pltpu.VMEM((2, 8, 1, 8, 128), jnp.float32),
            pltpu.SemaphoreType.DMA,
            pltpu.SemaphoreType.DMA((2,)),
            pltpu.SemaphoreType.DMA((2,)),
        ],
        compiler_params=pltpu.CompilerParams(
            use_tc_tiling_on_sc=False, needs_layout_passes=False),
    )


@jax.jit
def _emb_call(ids_t, embt):
    tlin = _make_transpose_kernel()(embt)
    table = tlin.reshape(TBL_PAD, DIM)
    return _make_gather_kernel()(ids_t, table)


def kernel(token_ids, embedding):
    embt = embedding.T
    ids_t = token_ids.T.astype(jnp.int32)
    out5d = _emb_call(ids_t, embt)
    return out5d.transpose(2, 4, 0, 1, 3).reshape(BATCH, SEQ, DIM)


# speedup vs baseline: 1.7543x; 1.0698x over previous
import jax
import jax.numpy as jnp
from jax import lax
from jax.experimental import pallas as pl
from jax.experimental.pallas import tpu as pltpu
from jax.experimental.pallas import tpu_sc as plsc

NUM_EMB = 1_000_000
DIM = 64
BATCH = 4096
SEQ = 200

NC = 2
NS = 16
NW = NC * NS

NBLK_A = 7813
TBL_PAD = NBLK_A * 128
BT = BATCH // 128


def _transpose_body(embt_hbm, tlin_hbm, in_v, out_v, isem, osem):
    wid = lax.axis_index("s") * NC + lax.axis_index("c")
    k16 = lax.iota(jnp.int32, 16)
    nj = (NBLK_A - wid + NW - 1) // NW

    def fire_in(i, slot):
        j = pl.multiple_of((wid + i * NW) * 128, 128)
        pltpu.async_copy(embt_hbm.at[pl.ds(0, DIM), pl.ds(j, 128)],
                         in_v.at[slot], isem.at[slot])

    def wait_in(i, slot):
        j = pl.multiple_of((wid + i * NW) * 128, 128)
        pltpu.make_async_copy(embt_hbm.at[pl.ds(0, DIM), pl.ds(j, 128)],
                              in_v.at[slot], isem.at[slot]).wait()

    def fire_out(i, slot):
        j = wid + i * NW
        pltpu.async_copy(out_v.at[slot],
                         tlin_hbm.at[pl.ds(j * 128 * DIM, 128 * DIM)],
                         osem.at[slot])

    def wait_out(i, slot):
        j = wid + i * NW
        pltpu.make_async_copy(out_v.at[slot],
                              tlin_hbm.at[pl.ds(j * 128 * DIM, 128 * DIM)],
                              osem.at[slot]).wait()

    sconst = [jnp.full((16,), b, jnp.int32) for b in range(4)]
    din_c = [[d0 + (k16 + s) % 16 for s in range(16)] for d0 in (0, 16, 32, 48)]
    oix_c = [[k16 * DIM + d0 + (k16 + s) % 16 for s in range(16)]
             for d0 in (0, 16, 32, 48)]

    for pb in range(3):
        @pl.when(pb < nj)
        def _():
            fire_in(pb, pb)

    def step(i, carry):
        slot = lax.rem(i, 4)
        sv = jnp.full((16,), slot, jnp.int32)

        @pl.when(i + 3 < nj)
        def _():
            fire_in(i + 3, lax.rem(i + 3, 4))

        wait_in(i, slot)

        @pl.when(i >= 4)
        def _():
            wait_out(i - 4, slot)

        def shuf(rb, c):
            rvec = rb * 16 + k16
            r064 = rb * (16 * DIM)
            for d0 in range(4):
                for s in range(16):
                    v = plsc.load_gather(in_v, [sv, din_c[d0][s], rvec])
                    plsc.store_scatter(out_v, [sv, oix_c[d0][s] + r064], v)
            return c

        lax.fori_loop(0, 8, shuf, 0)
        fire_out(i, slot)
        return carry

    lax.fori_loop(0, nj, step, 0)

    def drain(q, carry):
        i = nj - 4 + q

        @pl.when(i >= 0)
        def _():
            wait_out(i, lax.rem(i, 4))
        return carry

    lax.fori_loop(0, 4, drain, 0)


def _gather_body(ids_hbm, table_hbm, out_hbm, idx_v, g_v, t_v, xsem, gsem, ssem):
    wid = lax.axis_index("s") * NC + lax.axis_index("c")
    k16 = lax.iota(jnp.int32, 16)

    pltpu.async_copy(ids_hbm.at[pl.ds(0, SEQ), pl.ds(wid * 128, 128)],
                     idx_v, xsem)
    pltpu.make_async_copy(ids_hbm.at[pl.ds(0, SEQ), pl.ds(wid * 128, 128)],
                          idx_v, xsem).wait()

    def fire_g(s, slot):
        pltpu.async_copy(table_hbm.at[idx_v.at[s]], g_v.at[slot], gsem.at[slot])

    def wait_g(s, slot):
        pltpu.make_async_copy(table_hbm.at[idx_v.at[s]], g_v.at[slot],
                              gsem.at[slot]).wait()

    def fire_s(s, slot):
        pltpu.async_copy(t_v.at[slot], out_hbm.at[s, pl.ds(0, 8), pl.ds(wid, 1)],
                         ssem.at[slot])

    def wait_s(s, slot):
        pltpu.make_async_copy(t_v.at[slot],
                              out_hbm.at[s, pl.ds(0, 8), pl.ds(wid, 1)],
                              ssem.at[slot]).wait()

    sconst = [jnp.full((16,), b, jnp.int32) for b in range(4)]
    zconst = jnp.full((16,), 0, jnp.int32)
    dvec_c = [[d0 + (k16 + s) % 16 for s in range(16)] for d0 in (0, 16, 32, 48)]
    dt_c = [[v // 8 for v in row] for row in dvec_c]
    di_c = [[v % 8 for v in row] for row in dvec_c]

    def shuffle(slot):
        def shuf(bb, c):
            bivec = bb * 16 + k16
            for d0 in range(4):
                for s in range(16):
                    v = plsc.load_gather(g_v, [sconst[slot], bivec, dvec_c[d0][s]])
                    plsc.store_scatter(
                        t_v, [sconst[slot], dt_c[d0][s], zconst, di_c[d0][s], bivec], v)
            return c
        lax.fori_loop(0, 8, shuf, 0)

    for pb in range(3):
        fire_g(pb, pb)
    ns4 = SEQ // 4

    def step(s4, carry):
        for b in range(4):
            s = 4 * s4 + b

            @pl.when(s + 3 < SEQ)
            def _():
                fire_g(s + 3, (b + 3) % 4)

            wait_g(s, b)

            @pl.when(s >= 4)
            def _():
                wait_s(s - 4, b)

            shuffle(b)
            fire_s(s, b)
        return carry

    lax.fori_loop(0, ns4, step, 0)
    for q in range(4):
        wait_s(SEQ - 4 + q, q % 4)


def _make_transpose_kernel():
    mesh = plsc.VectorSubcoreMesh(core_axis_name="c", subcore_axis_name="s")
    return pl.kernel(
        _transpose_body,
        out_type=jax.ShapeDtypeStruct((TBL_PAD * DIM,), jnp.float32),
        mesh=mesh,
        scratch_types=[
            pltpu.VMEM((4, DIM, 128), jnp.float32),
            pltpu.VMEM((4, 128 * DIM), jnp.float32),
            pltpu.SemaphoreType.DMA((4,)),
            pltpu.SemaphoreType.DMA((4,)),
        ],
        compiler_params=pltpu.CompilerParams(
            use_tc_tiling_on_sc=True, needs_layout_passes=False,
            disable_bounds_checks=True),
    )


def _make_gather_kernel():
    mesh = plsc.VectorSubcoreMesh(core_axis_name="c", subcore_axis_name="s")
    return pl.kernel(
        _gather_body,
        out_type=jax.ShapeDtypeStruct((SEQ, 8, BT, 8, 128), jnp.float32),
        mesh=mesh,
        scratch_types=[
            pltpu.VMEM((SEQ, 128), jnp.int32),
            pltpu.VMEM((4, 128, DIM), jnp.float32),
            pltpu.VMEM((4, 8, 1, 8, 128), jnp.float32),
            pltpu.SemaphoreType.DMA,
            pltpu.SemaphoreType.DMA((4,)),
            pltpu.SemaphoreType.DMA((4,)),
        ],
        compiler_params=pltpu.CompilerParams(
            use_tc_tiling_on_sc=False, needs_layout_passes=False),
    )


@jax.jit
def _emb_call(ids_t, embt):
    tlin = _make_transpose_kernel()(embt)
    table = tlin.reshape(TBL_PAD, DIM)
    return _make_gather_kernel()(ids_t, table)


def kernel(token_ids, embedding):
    embt = embedding.T
    ids_t = token_ids.T.astype(jnp.int32)
    out5d = _emb_call(ids_t, embt)
    return out5d.transpose(2, 4, 0, 1, 3).reshape(BATCH, SEQ, DIM)
